# Initial kernel scaffold; baseline (speedup 1.0000x reference)
import jax
import jax.numpy as jnp
from jax.experimental import pallas as pl


def kernel(x, gold):
    raise NotImplementedError("write your pallas kernel here")


# speedup vs baseline: 1.4925x; 1.4925x over previous
import functools

import jax
import jax.numpy as jnp
from jax import lax
from jax.experimental import pallas as pl
from jax.experimental.pallas import tpu as pltpu
from jax.experimental.pallas import tpu_sc as plsc

MARGIN = 0.1
B, V = 1024, 100000

_NC, _NS, _L = 2, 16, 16
_NW = _NC * _NS
_BPW = B // _NW

@functools.cache
def _gather_gold_kernel():
    mesh = plsc.VectorSubcoreMesh(core_axis_name="c", subcore_axis_name="s")

    @functools.partial(
        pl.kernel,
        mesh=mesh,
        out_type=jax.ShapeDtypeStruct((B,), jnp.float32),
        scratch_types=[
            pltpu.VMEM((_BPW,), jnp.int32),
            pltpu.VMEM((_BPW,), jnp.int32),
            pltpu.VMEM((_BPW,), jnp.float32),
            pltpu.SemaphoreType.DMA,
        ],
    )
    def _gather_gold(xflat_hbm, gold_hbm, out_hbm, gold_v, idx_v, gs_v, sem):
        wid = lax.axis_index("s") * _NC + lax.axis_index("c")
        base = wid * _BPW
        pltpu.sync_copy(gold_hbm.at[pl.ds(base, _BPW)], gold_v)
        for j in range(_BPW // _L):
            rows = base + j * _L + lax.iota(jnp.int32, _L)
            idx_v[pl.ds(j * _L, _L)] = rows * V + gold_v[pl.ds(j * _L, _L)]
        pltpu.async_copy(xflat_hbm.at[idx_v], gs_v, sem).wait()
        pltpu.sync_copy(gs_v, out_hbm.at[pl.ds(base, _BPW)])

    return _gather_gold


def _loss_body(gs_ref, x_ref, o_ref):
    i = pl.program_id(0)
    xv = x_ref[...]
    g = gs_ref[...]
    m = xv > (g - MARGIN)
    cnt = jnp.sum(m.astype(jnp.float32), axis=1, keepdims=True) - 1.0
    s = jnp.sum(jnp.where(m, xv, 0.0), axis=1, keepdims=True) - g
    denom = jnp.maximum(cnt, 1.0)
    loss = jnp.where(cnt > 0.0, (s + cnt * (MARGIN - g)) / denom, 0.0)
    part = (jnp.sum(loss) / B).reshape(1, 1)

    @pl.when(i == 0)
    def _():
        o_ref[...] = jnp.zeros_like(o_ref)

    o_ref[...] += part


_BR = 32


def _loss_call(gs2d, x):
    grid = B // _BR
    return pl.pallas_call(
        _loss_body,
        grid=(grid,),
        in_specs=[
            pl.BlockSpec((_BR, 1), lambda i: (i, 0)),
            pl.BlockSpec((_BR, V), lambda i: (i, 0)),
        ],
        out_specs=pl.BlockSpec((1, 1), lambda i: (0, 0)),
        out_shape=jax.ShapeDtypeStruct((1, 1), jnp.float32),
        compiler_params=pltpu.CompilerParams(
            dimension_semantics=("arbitrary",),
        ),
    )(gs2d, x)


def kernel(x, gold):
    gold = gold.astype(jnp.int32)
    gs = _gather_gold_kernel()(x.reshape(B * V), gold)
    out = _loss_call(gs.reshape(B, 1), x)
    return out.reshape(())


# speedup vs baseline: 3.1340x; 2.0998x over previous
import functools

import jax
import jax.numpy as jnp
from jax import lax
from jax.experimental import pallas as pl
from jax.experimental.pallas import tpu as pltpu

MARGIN = 0.1
B, V = 1024, 100000


def _loss_body(gold_ref, x_ref, o_ref):
    i = pl.program_id(0)
    xv = x_ref[...]
    gold = gold_ref[...]
    col = lax.broadcasted_iota(jnp.int32, xv.shape, 1)
    g = jnp.sum(jnp.where(col == gold, xv, 0.0), axis=1, keepdims=True)
    m = xv > (g - MARGIN)
    cnt = jnp.sum(m.astype(jnp.float32), axis=1, keepdims=True) - 1.0
    s = jnp.sum(jnp.where(m, xv, 0.0), axis=1, keepdims=True) - g
    denom = jnp.maximum(cnt, 1.0)
    loss = jnp.where(cnt > 0.0, (s + cnt * (MARGIN - g)) / denom, 0.0)
    part = (jnp.sum(loss) / B).reshape(1, 1)

    @pl.when(i == 0)
    def _():
        o_ref[...] = jnp.zeros_like(o_ref)

    o_ref[...] += part


_BR = 32


def _loss_call(gold2d, x):
    grid = B // _BR
    return pl.pallas_call(
        _loss_body,
        grid=(grid,),
        in_specs=[
            pl.BlockSpec((_BR, 1), lambda i: (i, 0)),
            pl.BlockSpec((_BR, V), lambda i: (i, 0)),
        ],
        out_specs=pl.BlockSpec((1, 1), lambda i: (0, 0)),
        out_shape=jax.ShapeDtypeStruct((1, 1), jnp.float32),
        compiler_params=pltpu.CompilerParams(
            dimension_semantics=("arbitrary",),
        ),
    )(gold2d, x)


def kernel(x, gold):
    gold = gold.astype(jnp.int32)
    out = _loss_call(gold.reshape(B, 1), x)
    return out.reshape(())
